# baseline (speedup 1.0000x reference)
import jax
import jax.numpy as jnp
from jax.experimental import pallas as pl
from jax.experimental.pallas import tpu as pltpu

_N = 8192
_B = 32
_K = 128
_M = 128
_TOPK = 5
_C = _N // 2
_NT = (((1,), (1,)), ((), ()))
_LOG_EPS = -36.8413614879047


def _gcl_kernel(kb_ref, k_ref, beta_ref, gamma_ref, a_ref, ct_ref,
                out_ref, scratch_ref, top5_ref, acc1_ref, acc2_ref, sums_ref):
    i = pl.program_id(0)

    @pl.when(i < 2)
    def _phase_a():
        k = k_ref[:, :]
        beta = beta_ref[:, :]
        rk = jnp.sqrt(jnp.sum(k * k, axis=1, keepdims=True))
        kb = kb_ref[:, :]
        scores = jax.lax.dot_general(k, kb, _NT,
                                     preferred_element_type=jnp.float32)
        ones = jnp.ones((1, _K), dtype=jnp.float32)
        rn2 = jax.lax.dot_general(ones, kb * kb, _NT,
                                  preferred_element_type=jnp.float32)
        denom = jnp.maximum(jnp.sqrt(rn2) * rk, 1e-8)
        logits = beta * (scores / denom)
        scratch_ref[:, pl.ds(i * _C, _C)] = logits
        cur = logits
        ms = []
        for _ in range(_TOPK):
            mv = jnp.max(cur, axis=1, keepdims=True)
            ms.append(mv)
            cur = jnp.where(cur == mv, -jnp.inf, cur)
        pad = jnp.full((_B, 128 - _TOPK), -jnp.inf, dtype=jnp.float32)
        top5_ref[:, pl.ds(i * 128, 128)] = jnp.concatenate(ms + [pad], axis=1)

    def _chunk(c):
        cur = top5_ref[:, :]
        t5 = None
        for _ in range(_TOPK):
            t5 = jnp.max(cur, axis=1, keepdims=True)
            cur = jnp.where(cur == t5, -jnp.inf, cur)
        gamma = gamma_ref[:, :]
        lg = scratch_ref[:, pl.ds(c * _C, _C)]
        u = jnp.exp(gamma * (lg + jnp.where(lg >= t5, 0.0, _LOG_EPS)))
        u2 = u * u
        ct = ct_ref[:, :]
        p1 = jnp.dot(u, ct, preferred_element_type=jnp.float32)
        p2 = jnp.dot(u2, ct, preferred_element_type=jnp.float32)
        su = jnp.sum(u, axis=1, keepdims=True)
        su2 = jnp.sum(u2, axis=1, keepdims=True)
        return p1, p2, su, su2

    @pl.when(i == 1)
    def _phase_b0():
        p1, p2, su, su2 = _chunk(0)
        acc1_ref[:, :] = p1
        acc2_ref[:, :] = p2
        sums_ref[:, 0:1] = su
        sums_ref[:, 1:2] = su2

    @pl.when(i == 2)
    def _phase_b1():
        p1, p2, su, su2 = _chunk(1)
        s1 = sums_ref[:, 0:1] + su
        s2 = sums_ref[:, 1:2] + su2
        winv = 1.0 / s1
        winv2 = winv * winv
        out_ref[:, :] = (winv * (acc1_ref[:, :] + p1)
                         - winv2 * (acc2_ref[:, :] + p2)
                         + (s2 * winv2) * a_ref[:, :])


def kernel(k, beta, g, s, gamma, a, a_k, content_bias, key_bias, candidates):
    del g, s, a_k, candidates
    return pl.pallas_call(
        _gcl_kernel,
        grid=(3,),
        in_specs=[
            pl.BlockSpec((_C, _K), lambda i: (jnp.minimum(i, 1), 0)),
            pl.BlockSpec((_B, _K), lambda i: (0, 0)),
            pl.BlockSpec((_B, 1), lambda i: (0, 0)),
            pl.BlockSpec((_B, 1), lambda i: (0, 0)),
            pl.BlockSpec((_B, _M), lambda i: (0, 0)),
            pl.BlockSpec((_C, _M), lambda i: (jnp.maximum(i - 1, 0), 0)),
        ],
        out_specs=pl.BlockSpec((_B, _M), lambda i: (0, 0)),
        out_shape=jax.ShapeDtypeStruct((_B, _M), jnp.float32),
        scratch_shapes=[
            pltpu.VMEM((_B, _N), jnp.float32),
            pltpu.VMEM((_B, 256), jnp.float32),
            pltpu.VMEM((_B, _M), jnp.float32),
            pltpu.VMEM((_B, _M), jnp.float32),
            pltpu.VMEM((_B, 8), jnp.float32),
        ],
        compiler_params=pltpu.CompilerParams(
            dimension_semantics=("arbitrary",)),
    )(key_bias, k, beta, gamma, a, content_bias)


# speedup vs baseline: 1.0119x; 1.0119x over previous
import jax
import jax.numpy as jnp
from jax.experimental import pallas as pl
from jax.experimental.pallas import tpu as pltpu

_N = 8192
_B = 32
_K = 128
_M = 128
_TOPK = 5
_C = _N // 2
_NT = (((1,), (1,)), ((), ()))
_LOG_EPS = -36.8413614879047


def _gcl_kernel(kb_ref, pk_ref, ct_ref, out_ref, scratch_ref):
    i = pl.program_id(0)

    @pl.when(i < 2)
    def _phase_a():
        k = pk_ref[:, 0:_K]
        beta = pk_ref[:, 256:257]
        rk = jnp.sqrt(jnp.sum(k * k, axis=1, keepdims=True))
        kb = kb_ref[:, :]
        scores = jax.lax.dot_general(k, kb, _NT,
                                     preferred_element_type=jnp.float32)
        ones = jnp.ones((1, _K), dtype=jnp.float32)
        rn2 = jax.lax.dot_general(ones, kb * kb, _NT,
                                  preferred_element_type=jnp.float32)
        denom = jnp.maximum(jnp.sqrt(rn2) * rk, 1e-8)
        scratch_ref[:, pl.ds(i * _C, _C)] = beta * (scores / denom)

    @pl.when(i == 1)
    def _weights():
        logits = scratch_ref[:, :]
        cur = logits
        t5 = None
        for _ in range(_TOPK):
            t5 = jnp.max(cur, axis=1, keepdims=True)
            cur = jnp.where(cur == t5, -jnp.inf, cur)
        gamma = pk_ref[:, 257:258]
        logf = jnp.where(logits >= t5, 0.0, _LOG_EPS)
        u = jnp.exp(gamma * (logits + logf))
        u2 = u * u
        winv = 1.0 / jnp.sum(u, axis=1, keepdims=True)
        sw2 = jnp.sum(u2, axis=1, keepdims=True) * (winv * winv)
        scratch_ref[:, :] = winv * u - (winv * winv) * u2
        out_ref[:, :] = sw2 * pk_ref[:, 128:256]

    @pl.when(i >= 1)
    def _phase_b():
        c = i - 1
        v = scratch_ref[:, pl.ds(c * _C, _C)]
        out_ref[:, :] += jnp.dot(v, ct_ref[:, :],
                                 preferred_element_type=jnp.float32)


def kernel(k, beta, g, s, gamma, a, a_k, content_bias, key_bias, candidates):
    del g, s, a_k, candidates
    packed = jnp.concatenate(
        [k, a, beta, gamma, jnp.zeros((_B, 126), jnp.float32)], axis=1)
    return pl.pallas_call(
        _gcl_kernel,
        grid=(3,),
        in_specs=[
            pl.BlockSpec((_C, _K), lambda i: (jnp.minimum(i, 1), 0)),
            pl.BlockSpec((_B, 384), lambda i: (0, 0)),
            pl.BlockSpec((_C, _M), lambda i: (jnp.maximum(i - 1, 0), 0)),
        ],
        out_specs=pl.BlockSpec((_B, _M), lambda i: (0, 0)),
        out_shape=jax.ShapeDtypeStruct((_B, _M), jnp.float32),
        scratch_shapes=[pltpu.VMEM((_B, _N), jnp.float32)],
        compiler_params=pltpu.CompilerParams(
            dimension_semantics=("arbitrary",)),
    )(key_bias, packed, content_bias)


# speedup vs baseline: 1.0601x; 1.0476x over previous
import jax
import jax.numpy as jnp
from jax.experimental import pallas as pl
from jax.experimental.pallas import tpu as pltpu

_N = 8192
_B = 32
_K = 128
_M = 128
_TOPK = 5
_C = _N // 2
_NT = (((1,), (1,)), ((), ()))
_LOG_EPS = -36.8413614879047


def _gcl_kernel(kb_ref, k_ref, beta_ref, gamma_ref, a_ref,
                ct_ref, out_ref, scratch_ref):
    i = pl.program_id(0)

    @pl.when(i < 2)
    def _phase_a():
        k = k_ref[:, :]
        beta = beta_ref[:, :]
        rk = jnp.sqrt(jnp.sum(k * k, axis=1, keepdims=True))
        kb = kb_ref[:, :]
        scores = jax.lax.dot_general(k, kb, _NT,
                                     preferred_element_type=jnp.float32)
        ones = jnp.ones((1, _K), dtype=jnp.float32)
        rn2 = jax.lax.dot_general(ones, kb * kb, _NT,
                                  preferred_element_type=jnp.float32)
        denom = jnp.maximum(jnp.sqrt(rn2) * rk, 1e-8)
        scratch_ref[:, pl.ds(i * _C, _C)] = beta * (scores / denom)

    @pl.when(i == 1)
    def _weights():
        logits = scratch_ref[:, :]
        cur = logits
        t5 = None
        for _ in range(_TOPK):
            t5 = jnp.max(cur, axis=1, keepdims=True)
            cur = jnp.where(cur == t5, -jnp.inf, cur)
        gamma = gamma_ref[:, :]
        logf = jnp.where(logits >= t5, 0.0, _LOG_EPS)
        u = jnp.exp(gamma * (logits + logf))
        u2 = u * u
        ones_n = jnp.ones((_N, 1), dtype=jnp.float32)
        s1 = jnp.dot(u, ones_n, preferred_element_type=jnp.float32)
        s2 = jnp.dot(u2, ones_n, preferred_element_type=jnp.float32)
        winv = 1.0 / s1
        winv2 = winv * winv
        scratch_ref[:, :] = winv * u - winv2 * u2
        out_ref[:, :] = (s2 * winv2) * a_ref[:, :]

    @pl.when(i >= 1)
    def _phase_b():
        c = i - 1
        v = scratch_ref[:, pl.ds(c * _C, _C)]
        out_ref[:, :] += jnp.dot(v, ct_ref[:, :],
                                 preferred_element_type=jnp.float32)


def kernel(k, beta, g, s, gamma, a, a_k, content_bias, key_bias, candidates):
    del g, s, a_k, candidates
    return pl.pallas_call(
        _gcl_kernel,
        grid=(3,),
        in_specs=[
            pl.BlockSpec((_C, _K), lambda i: (jnp.minimum(i, 1), 0)),
            pl.BlockSpec((_B, _K), lambda i: (0, 0)),
            pl.BlockSpec((_B, 1), lambda i: (0, 0)),
            pl.BlockSpec((_B, 1), lambda i: (0, 0)),
            pl.BlockSpec((_B, _M), lambda i: (0, 0)),
            pl.BlockSpec((_C, _M), lambda i: (jnp.maximum(i - 1, 0), 0)),
        ],
        out_specs=pl.BlockSpec((_B, _M), lambda i: (0, 0)),
        out_shape=jax.ShapeDtypeStruct((_B, _M), jnp.float32),
        scratch_shapes=[pltpu.VMEM((_B, _N), jnp.float32)],
        compiler_params=pltpu.CompilerParams(
            dimension_semantics=("arbitrary",)),
    )(key_bias, k, beta, gamma, a, content_bias)


# speedup vs baseline: 1.0983x; 1.0361x over previous
import jax
import jax.numpy as jnp
from jax.experimental import pallas as pl
from jax.experimental.pallas import tpu as pltpu

_N = 8192
_B = 32
_K = 128
_M = 128
_TOPK = 5
_C = _N // 2
_NT = (((1,), (1,)), ((), ()))
_LOG_EPS = -36.8413614879047


def _gcl_kernel(kb_ref, k_ref, beta_ref, gamma_ref, a_ref,
                ct_ref, out_ref, scratch_ref):
    i = pl.program_id(0)

    @pl.when(i < 2)
    def _phase_a():
        k = k_ref[:, :]
        beta = beta_ref[:, :]
        rk = jnp.sqrt(jnp.sum(k * k, axis=1, keepdims=True))
        kb = kb_ref[:, :]
        scores = jax.lax.dot_general(k, kb, _NT,
                                     preferred_element_type=jnp.float32)
        ones = jnp.ones((1, _K), dtype=jnp.float32)
        rn2 = jax.lax.dot_general(ones, kb * kb, _NT,
                                  preferred_element_type=jnp.float32)
        denom = jnp.maximum(jnp.sqrt(rn2) * rk, 1e-8)
        scratch_ref[:, pl.ds(i * _C, _C)] = beta * (scores / denom)

    @pl.when(i == 1)
    def _weights():
        logits = scratch_ref[:, :]
        cur = logits
        t5 = None
        for _ in range(_TOPK):
            t5 = jnp.max(cur, axis=1, keepdims=True)
            cur = jnp.where(cur == t5, -jnp.inf, cur)
        gamma = gamma_ref[:, :]
        logf = jnp.where(logits >= t5, 0.0, _LOG_EPS)
        u = jnp.exp(gamma * (logits + logf))
        u2 = u * u
        winv = 1.0 / jnp.sum(u, axis=1, keepdims=True)
        sw2 = jnp.sum(u2, axis=1, keepdims=True) * (winv * winv)
        scratch_ref[:, :] = winv * u - (winv * winv) * u2
        out_ref[:, :] = sw2 * a_ref[:, :]

    @pl.when(i >= 1)
    def _phase_b():
        c = i - 1
        v = scratch_ref[:, pl.ds(c * _C, _C)]
        out_ref[:, :] += jnp.dot(v, ct_ref[:, :],
                                 preferred_element_type=jnp.float32)


def kernel(k, beta, g, s, gamma, a, a_k, content_bias, key_bias, candidates):
    del g, s, a_k, candidates
    return pl.pallas_call(
        _gcl_kernel,
        grid=(3,),
        in_specs=[
            pl.BlockSpec((_C, _K), lambda i: (jnp.minimum(i, 1), 0)),
            pl.BlockSpec((_B, _K), lambda i: (0, 0)),
            pl.BlockSpec((_B, 1), lambda i: (0, 0)),
            pl.BlockSpec((_B, 1), lambda i: (0, 0)),
            pl.BlockSpec((_B, _M), lambda i: (0, 0)),
            pl.BlockSpec((_C, _M), lambda i: (jnp.maximum(i - 1, 0), 0)),
        ],
        out_specs=pl.BlockSpec((_B, _M), lambda i: (0, 0)),
        out_shape=jax.ShapeDtypeStruct((_B, _M), jnp.float32),
        scratch_shapes=[pltpu.VMEM((_B, _N), jnp.float32)],
        compiler_params=pltpu.CompilerParams(
            dimension_semantics=("arbitrary",)),
    )(key_bias, k, beta, gamma, a, content_bias)
